# baseline (speedup 1.0000x reference)
import jax
import jax.numpy as jnp
from jax.experimental import pallas as pl

CB = 128
D = 32
BLOCK = 8192
SEQ = 8


def _vq_body(lat_ref, cb_ref, lane_ref, q_ref, p_ref):
    x = lat_ref[...]
    cb = cb_ref[...]
    s = jnp.sum(x * x, axis=1, keepdims=True)
    n = jnp.sum(cb * cb, axis=1)[None, :]
    mm2 = jax.lax.dot_general(x, -2.0 * cb, (((1,), (1,)), ((), ())),
                              preferred_element_type=jnp.float32)
    d = (s + n) + mm2
    dmin = jnp.min(d, axis=1, keepdims=True)
    lane = lane_ref[...]
    idx = jnp.min(jnp.where(d == dmin, lane, float(CB)), axis=1, keepdims=True)
    oh = (lane == idx).astype(jnp.float32)
    q = jax.lax.dot_general(oh, cb, (((1,), (0,)), ((), ())),
                            preferred_element_type=jnp.float32)
    q_ref[...] = q
    p_ref[...] = q


def kernel(latent, codebook):
    B = latent.shape[0]
    rows = B * SEQ
    cbs = jnp.broadcast_to(codebook[None], (B, CB, D))
    lat2 = latent.reshape(rows, D)
    grid = rows // BLOCK
    q, p = pl.pallas_call(
        _vq_body,
        grid=(grid,),
        in_specs=[
            pl.BlockSpec((BLOCK, D), lambda i: (i, 0)),
            pl.BlockSpec((CB, D), lambda i: (0, 0)),
            pl.BlockSpec((1, CB), lambda i: (0, 0)),
        ],
        out_specs=[
            pl.BlockSpec((BLOCK, D), lambda i: (i, 0)),
            pl.BlockSpec((BLOCK, D), lambda i: (i, 0)),
        ],
        out_shape=[
            jax.ShapeDtypeStruct((rows, D), jnp.float32),
            jax.ShapeDtypeStruct((rows, D), jnp.float32),
        ],
    )(lat2, codebook, jnp.arange(CB, dtype=jnp.float32).reshape(1, CB))
    shape = latent.shape
    return (p.reshape(shape), q.reshape(shape), cbs)


# speedup vs baseline: 1.1275x; 1.1275x over previous
import jax
import jax.numpy as jnp
from jax.experimental import pallas as pl

CB = 128
D = 32
BLOCK = 8192
SEQ = 8


def _vq_body(lat_ref, cb_ref, lane_ref, q_ref):
    x = lat_ref[...]
    cb = cb_ref[...]
    s = jnp.sum(x * x, axis=1, keepdims=True)
    n = jnp.sum(cb * cb, axis=1)[None, :]
    mm2 = jax.lax.dot_general(x, -2.0 * cb, (((1,), (1,)), ((), ())),
                              preferred_element_type=jnp.float32)
    d = (s + n) + mm2
    dmin = jnp.min(d, axis=1, keepdims=True)
    lane = lane_ref[...]
    idx = jnp.min(jnp.where(d == dmin, lane, float(CB)), axis=1, keepdims=True)
    oh = (lane == idx).astype(jnp.float32)
    q = jax.lax.dot_general(oh, cb, (((1,), (0,)), ((), ())),
                            preferred_element_type=jnp.float32)
    q_ref[...] = q


def kernel(latent, codebook):
    B = latent.shape[0]
    rows = B * SEQ
    cbs = jnp.broadcast_to(codebook[None], (B, CB, D))
    lat2 = latent.reshape(rows, D)
    grid = rows // BLOCK
    q = pl.pallas_call(
        _vq_body,
        grid=(grid,),
        in_specs=[
            pl.BlockSpec((BLOCK, D), lambda i: (i, 0)),
            pl.BlockSpec((CB, D), lambda i: (0, 0)),
            pl.BlockSpec((1, CB), lambda i: (0, 0)),
        ],
        out_specs=pl.BlockSpec((BLOCK, D), lambda i: (i, 0)),
        out_shape=jax.ShapeDtypeStruct((rows, D), jnp.float32),
    )(lat2, codebook, jnp.arange(CB, dtype=jnp.float32).reshape(1, CB))
    shape = latent.shape
    qr = q.reshape(shape)
    return (qr, qr, cbs)
